# baseline (speedup 1.0000x reference)
import jax
import jax.numpy as jnp
from jax import lax
from jax.experimental import pallas as pl
from jax.experimental.pallas import tpu as pltpu

B = 8
N = 2048
R = 512
NB = N // R


def _body(s_ref, m_ref, out_ref, sums_ref, cnts_ref):
    b = pl.program_id(0)
    i = pl.program_id(1)

    s = s_ref[0]
    m = m_ref[0] != 0
    part_sum = jnp.sum(jnp.where(m, s, 0.0))
    part_cnt = jnp.sum(m.astype(jnp.float32))

    @pl.when(i == 0)
    def _init():
        sums_ref[b] = part_sum
        cnts_ref[b] = part_cnt

    @pl.when(i != 0)
    def _acc():
        sums_ref[b] = sums_ref[b] + part_sum
        cnts_ref[b] = cnts_ref[b] + part_cnt

    @pl.when((b == B - 1) & (i == NB - 1))
    def _fin():
        acc = 0.0
        for bb in range(B):
            acc += sums_ref[bb] / cnts_ref[bb]
        out_ref[0, 0] = -acc / B


def kernel(scores, assigns):
    masks = assigns.view(jnp.int8)
    out = pl.pallas_call(
        _body,
        grid=(B, NB),
        in_specs=[
            pl.BlockSpec((1, R, N), lambda b, i: (b, i, 0)),
            pl.BlockSpec((1, R, N), lambda b, i: (b, i, 0)),
        ],
        out_specs=pl.BlockSpec(
            (1, 1), lambda b, i: (0, 0), memory_space=pltpu.SMEM
        ),
        out_shape=jax.ShapeDtypeStruct((1, 1), jnp.float32),
        scratch_shapes=[
            pltpu.SMEM((B,), jnp.float32),
            pltpu.SMEM((B,), jnp.float32),
        ],
    )(scores, masks)
    return out[0, 0]


# speedup vs baseline: 1.0985x; 1.0985x over previous
import jax
import jax.numpy as jnp
from jax import lax
from jax.experimental import pallas as pl
from jax.experimental.pallas import tpu as pltpu

B = 8
N = 2048
R = 2048
NB = N // R


def _body(s_ref, m_ref, out_ref, sums_ref, cnts_ref):
    b = pl.program_id(0)
    i = pl.program_id(1)

    s = s_ref[0]
    m = m_ref[0] != 0
    part_sum = jnp.sum(jnp.where(m, s, 0.0))
    part_cnt = jnp.sum(m.astype(jnp.float32))

    @pl.when(i == 0)
    def _init():
        sums_ref[b] = part_sum
        cnts_ref[b] = part_cnt

    @pl.when(i != 0)
    def _acc():
        sums_ref[b] = sums_ref[b] + part_sum
        cnts_ref[b] = cnts_ref[b] + part_cnt

    @pl.when((b == B - 1) & (i == NB - 1))
    def _fin():
        acc = 0.0
        for bb in range(B):
            acc += sums_ref[bb] / cnts_ref[bb]
        out_ref[0, 0] = -acc / B


def kernel(scores, assigns):
    masks = assigns.view(jnp.int8)
    out = pl.pallas_call(
        _body,
        grid=(B, NB),
        in_specs=[
            pl.BlockSpec((1, R, N), lambda b, i: (b, i, 0)),
            pl.BlockSpec((1, R, N), lambda b, i: (b, i, 0)),
        ],
        out_specs=pl.BlockSpec(
            (1, 1), lambda b, i: (0, 0), memory_space=pltpu.SMEM
        ),
        out_shape=jax.ShapeDtypeStruct((1, 1), jnp.float32),
        scratch_shapes=[
            pltpu.SMEM((B,), jnp.float32),
            pltpu.SMEM((B,), jnp.float32),
        ],
    )(scores, masks)
    return out[0, 0]
